# baseline (speedup 1.0000x reference)
import numpy as np

import jax
import jax.numpy as jnp
from jax import lax
from jax.experimental import pallas as pl

STRIDE = 8
IMG_H = 512
IMG_W = 512
NA = 9
NCLS = 80
NB = 8
NH = IMG_H // STRIDE
NW = IMG_W // STRIDE
ROWS = NH * NW
NSPLIT = 4
CHUNK = ROWS // NSPLIT
GA = 3


def _anchor_wh_np():
    base = 4 * STRIDE
    scales = [2.0 ** 0.0, 2.0 ** (1.0 / 3.0), 2.0 ** (2.0 / 3.0)]
    ratios = [(1.0, 1.0), (1.4, 0.7), (0.7, 1.4)]
    anchors = [(base * sc * rt[0], base * sc * rt[1]) for sc in scales for rt in ratios]
    return np.array(anchors, dtype=np.float32)


def _cls_reduce(x):
    x2 = x.reshape(GA * CHUNK, NCLS)
    m = jnp.max(x2, axis=1, keepdims=True)
    rev = lax.broadcasted_iota(jnp.int32, (GA * CHUNK, NCLS), 1).astype(jnp.float32)
    picked = jnp.max(jnp.where(x2 == m, (NCLS - 1.0) - rev, -1.0),
                     axis=1, keepdims=True)
    return (m.reshape(GA, CHUNK // 128, 128),
            picked.reshape(GA, CHUNK // 128, 128))


def _retina_body(tab_ref, t_ref, c0_ref, c1_ref, c2_ref, c3_ref,
                 bbox_ref, idx_ref, score_ref):
    row = lax.broadcasted_iota(jnp.int32, (ROWS, 4), 0)
    comp = lax.broadcasted_iota(jnp.int32, (ROWS, 4), 1)
    wf = (row & (NW - 1)).astype(jnp.float32)
    hf = (row >> 6).astype(jnp.float32)
    off = jnp.where(comp == 0, wf * STRIDE + STRIDE / 2,
                    jnp.where(comp == 1, hf * STRIDE + STRIDE / 2, 0.0))
    hi = comp >= 2
    for g in range(GA):
        tg = t_ref[0, g]
        aw = tab_ref[0, g, 0]
        ah = tab_ref[0, g, 1]
        scale = jnp.where((comp & 1) == 0, aw, ah)
        val = jnp.where(hi, jnp.exp(tg) * scale, off + tg * scale)
        bbox_ref[0, g] = jnp.clip(val, 1.0, float(max(IMG_H, IMG_W)))

    parts = [_cls_reduce(ref[0]) for ref in (c0_ref, c1_ref, c2_ref, c3_ref)]
    m2 = jnp.concatenate([p[0] for p in parts], axis=1)
    picked2 = jnp.concatenate([p[1] for p in parts], axis=1)
    idx_ref[0] = ((NCLS - 1.0) - picked2).astype(jnp.int32)
    score_ref[0] = jax.nn.sigmoid(m2)


def kernel(t_xywh, cls_logits):
    t = t_xywh.reshape(NB, NA, ROWS, 4)
    cls = cls_logits.reshape(NB, NA, ROWS, NCLS)
    tab = jnp.asarray(_anchor_wh_np()).reshape(NA // GA, GA, 2)

    def _cls_spec(k):
        return pl.BlockSpec((1, GA, CHUNK, NCLS), lambda b, g, k=k: (b, g, k, 0))

    bbox, idx, score = pl.pallas_call(
        _retina_body,
        grid=(NB, NA // GA),
        in_specs=[
            pl.BlockSpec((1, GA, 2), lambda b, g: (g, 0, 0)),
            pl.BlockSpec((1, GA, ROWS, 4), lambda b, g: (b, g, 0, 0)),
            _cls_spec(0), _cls_spec(1), _cls_spec(2), _cls_spec(3),
        ],
        out_specs=[
            pl.BlockSpec((1, GA, ROWS, 4), lambda b, g: (b, g, 0, 0)),
            pl.BlockSpec((1, GA, 32, 128), lambda b, g: (b, g, 0, 0)),
            pl.BlockSpec((1, GA, 32, 128), lambda b, g: (b, g, 0, 0)),
        ],
        out_shape=[
            jax.ShapeDtypeStruct((NB, NA, ROWS, 4), jnp.float32),
            jax.ShapeDtypeStruct((NB, NA, 32, 128), jnp.int32),
            jax.ShapeDtypeStruct((NB, NA, 32, 128), jnp.float32),
        ],
    )(tab, t, cls, cls, cls, cls)

    return (
        bbox.reshape(NB, NA * ROWS, 4),
        idx.reshape(NB, NA * ROWS),
        score.reshape(NB, NA * ROWS),
    )


# speedup vs baseline: 1.6254x; 1.6254x over previous
import numpy as np

import jax
import jax.numpy as jnp
from jax import lax
from jax.experimental import pallas as pl

STRIDE = 8
IMG_H = 512
IMG_W = 512
NA = 9
NCLS = 80
NB = 8
NH = IMG_H // STRIDE
NW = IMG_W // STRIDE
ROWS = NH * NW
NSPLIT = 4
CHUNK = ROWS // NSPLIT
GA = 3


def _scale_tab_np():
    base = 4 * STRIDE
    scales = [2.0 ** 0.0, 2.0 ** (1.0 / 3.0), 2.0 ** (2.0 / 3.0)]
    ratios = [(1.0, 1.0), (1.4, 0.7), (0.7, 1.4)]
    anchors = [(base * sc * rt[0], base * sc * rt[1]) for sc in scales for rt in ratios]
    awh = np.array(anchors, dtype=np.float32)
    tab = np.empty((NA, NW * 4), dtype=np.float32)
    tab[:, 0::2] = awh[:, 0:1]
    tab[:, 1::2] = awh[:, 1:2]
    return tab


def _cls_reduce(x):
    x2 = x.reshape(GA * CHUNK, NCLS)
    m = jnp.max(x2, axis=1, keepdims=True)
    rev = lax.broadcasted_iota(jnp.int32, (GA * CHUNK, NCLS), 1).astype(jnp.float32)
    picked = jnp.max(jnp.where(x2 == m, (NCLS - 1.0) - rev, -1.0),
                     axis=1, keepdims=True)
    return (m.reshape(GA, CHUNK // 128, 128),
            picked.reshape(GA, CHUNK // 128, 128))


def _retina_body(tab_ref, t_ref, c0_ref, c1_ref, c2_ref, c3_ref,
                 bbox_ref, idx_ref, score_ref):
    tt = t_ref[0]
    shp = (GA, NH, NW * 4)
    c = lax.broadcasted_iota(jnp.int32, shp, 2)
    h = lax.broadcasted_iota(jnp.int32, shp, 1)
    comp = c & 3
    wf = (c >> 2).astype(jnp.float32)
    hf = h.astype(jnp.float32)
    scale = tab_ref[0][:, None, :]
    off = jnp.where(comp == 0, wf * STRIDE + STRIDE / 2,
                    jnp.where(comp == 1, hf * STRIDE + STRIDE / 2, 0.0))
    val = jnp.where(comp >= 2, jnp.exp(tt) * scale, off + tt * scale)
    bbox_ref[0] = jnp.clip(val, 1.0, float(max(IMG_H, IMG_W)))

    parts = [_cls_reduce(ref[0]) for ref in (c0_ref, c1_ref, c2_ref, c3_ref)]
    m2 = jnp.concatenate([p[0] for p in parts], axis=1)
    picked2 = jnp.concatenate([p[1] for p in parts], axis=1)
    idx_ref[0] = ((NCLS - 1.0) - picked2).astype(jnp.int32)
    score_ref[0] = jax.nn.sigmoid(m2)


def kernel(t_xywh, cls_logits):
    t = t_xywh.reshape(NB, NA, NH, NW * 4)
    cls = cls_logits.reshape(NB, NA, ROWS, NCLS)
    tab = jnp.asarray(_scale_tab_np()).reshape(NA // GA, GA, NW * 4)

    def _cls_spec(k):
        return pl.BlockSpec((1, GA, CHUNK, NCLS), lambda b, g, k=k: (b, g, k, 0))

    bbox, idx, score = pl.pallas_call(
        _retina_body,
        grid=(NB, NA // GA),
        in_specs=[
            pl.BlockSpec((1, GA, NW * 4), lambda b, g: (g, 0, 0)),
            pl.BlockSpec((1, GA, NH, NW * 4), lambda b, g: (b, g, 0, 0)),
            _cls_spec(0), _cls_spec(1), _cls_spec(2), _cls_spec(3),
        ],
        out_specs=[
            pl.BlockSpec((1, GA, NH, NW * 4), lambda b, g: (b, g, 0, 0)),
            pl.BlockSpec((1, GA, 32, 128), lambda b, g: (b, g, 0, 0)),
            pl.BlockSpec((1, GA, 32, 128), lambda b, g: (b, g, 0, 0)),
        ],
        out_shape=[
            jax.ShapeDtypeStruct((NB, NA, NH, NW * 4), jnp.float32),
            jax.ShapeDtypeStruct((NB, NA, 32, 128), jnp.int32),
            jax.ShapeDtypeStruct((NB, NA, 32, 128), jnp.float32),
        ],
    )(tab, t, cls, cls, cls, cls)

    return (
        bbox.reshape(NB, NA * ROWS, 4),
        idx.reshape(NB, NA * ROWS),
        score.reshape(NB, NA * ROWS),
    )
